# Initial kernel scaffold; baseline (speedup 1.0000x reference)
import functools

import jax
import jax.numpy as jnp
from jax import lax
from jax.experimental import pallas as pl
from jax.experimental.pallas import tpu as pltpu
from jax.experimental.pallas import tpu_sc as plsc

N_USERS = 30000
N_ITEMS = 70000
NN = N_USERS + N_ITEMS
HALF = NN // 2
EMB = 32
NEDGE = 1600000
EPAD = 1638400
ROWS2D = EPAD // 128
TROWS = ROWS2D // 16
WINDOWS = 100
ACC_ROWS = 51200
SLICE = ACC_ROWS // 16
N_LAYERS = 3

_mesh = plsc.VectorSubcoreMesh(core_axis_name="c", subcore_axis_name="s")


@functools.partial(
    pl.kernel,
    out_type=jax.ShapeDtypeStruct((NN,), jnp.float32),
    mesh=_mesh,
    scratch_types=[
        pltpu.VMEM_SHARED((ACC_ROWS,), jnp.float32),
        pltpu.VMEM((8, 128), jnp.int32),
        pltpu.VMEM((8, 128), jnp.int32),
        pltpu.VMEM((8, 128), jnp.float32),
        pltpu.VMEM((800,), jnp.float32),
    ],
)
def _deg_kernel(dst_hbm, deg_hbm, acc, dstb, ldst, ones, zbuf):
    c = lax.axis_index("c")
    t = lax.axis_index("s")
    base = c * HALF
    z16 = jnp.zeros((16,), jnp.float32)
    o16 = jnp.ones((16,), jnp.float32)

    def zfill(i, _):
        zbuf[pl.ds(i * 16, 16)] = z16
        return 0

    lax.fori_loop(0, 50, zfill, 0)
    for r in range(8):
        for q in range(8):
            ones[r, pl.ds(q * 16, 16)] = o16
    for k in range(4):
        pltpu.sync_copy(zbuf, acc.at[pl.ds(t * SLICE + k * 800, 800)])
    plsc.subcore_barrier()

    def body(w, _):
        row = t * TROWS + w * 8
        pltpu.sync_copy(dst_hbm.at[pl.ds(row, 8)], dstb)
        for r in range(8):
            for q in range(8):
                v = dstb[r, pl.ds(q * 16, 16)]
                inr = (v >= base) & (v < base + HALF)
                dump = (HALF + (r * 8 + q) * 16) + lax.iota(jnp.int32, 16)
                ldst[r, pl.ds(q * 16, 16)] = jnp.where(inr, v - base, dump)
        for r in range(8):
            pltpu.sync_copy(ones.at[r], acc.at[ldst.at[r]], add=True)
        return 0

    lax.fori_loop(0, WINDOWS, body, 0)
    plsc.subcore_barrier()

    @pl.when(t < 15)
    def _():
        for k in range(4):
            lo = t * SLICE + k * 800
            pltpu.sync_copy(acc.at[pl.ds(lo, 800)],
                            deg_hbm.at[pl.ds(base + lo, 800)])

    @pl.when(t == 15)
    def _():
        for k in range(2):
            lo = t * SLICE + k * 800
            pltpu.sync_copy(acc.at[pl.ds(lo, 800)],
                            deg_hbm.at[pl.ds(base + lo, 800)])
        lo = t * SLICE + 1600
        pltpu.sync_copy(acc.at[pl.ds(lo, 400)],
                        deg_hbm.at[pl.ds(base + lo, 400)])


@functools.partial(
    pl.kernel,
    out_type=(jax.ShapeDtypeStruct((NN, EMB), jnp.float32),
              jax.ShapeDtypeStruct((NN, EMB), jnp.float32)),
    mesh=_mesh,
    scratch_types=[
        pltpu.VMEM_SHARED((ACC_ROWS, EMB), jnp.float32),
        pltpu.VMEM((8, 128), jnp.int32),
        pltpu.VMEM((8, 128), jnp.int32),
        pltpu.VMEM((8, 128), jnp.int32),
        pltpu.VMEM((1024, EMB), jnp.float32),
        pltpu.VMEM((800, EMB), jnp.float32),
        pltpu.VMEM((800,), jnp.float32),
        pltpu.VMEM((800, EMB), jnp.float32),
        pltpu.VMEM((800, EMB), jnp.float32),
        pltpu.SemaphoreType.DMA,
    ],
)
def _prop_kernel(src_hbm, dst_hbm, zin_hbm, dinv_hbm, xout_hbm, zout_hbm,
                 acc, srcb, dstb, ldst, rows, abuf, dbuf, xbuf, zbuf, sem):
    c = lax.axis_index("c")
    t = lax.axis_index("s")
    base = c * HALF
    z16 = jnp.zeros((16,), jnp.float32)

    def zfill(i, _):
        abuf[i, pl.ds(0, 16)] = z16
        abuf[i, pl.ds(16, 16)] = z16
        return 0

    lax.fori_loop(0, 800, zfill, 0)
    for k in range(4):
        pltpu.sync_copy(abuf, acc.at[pl.ds(t * SLICE + k * 800, 800)])
    plsc.subcore_barrier()

    def body(w, _):
        row = t * TROWS + w * 8
        pltpu.sync_copy(src_hbm.at[pl.ds(row, 8)], srcb)
        pltpu.sync_copy(dst_hbm.at[pl.ds(row, 8)], dstb)
        handles = [
            pltpu.async_copy(zin_hbm.at[srcb.at[j]],
                             rows.at[pl.ds(j * 128, 128)], sem)
            for j in range(8)
        ]
        for r in range(8):
            for q in range(8):
                v = dstb[r, pl.ds(q * 16, 16)]
                inr = (v >= base) & (v < base + HALF)
                dump = (HALF + (r * 8 + q) * 16) + lax.iota(jnp.int32, 16)
                ldst[r, pl.ds(q * 16, 16)] = jnp.where(inr, v - base, dump)
        for h in handles:
            h.wait()
        for j in range(8):
            pltpu.sync_copy(rows.at[pl.ds(j * 128, 128)],
                            acc.at[ldst.at[j]], add=True)
        return 0

    lax.fori_loop(0, WINDOWS, body, 0)
    plsc.subcore_barrier()

    def chunk(lo, ch):
        g = base + lo
        pltpu.sync_copy(acc.at[pl.ds(lo, ch)], abuf.at[pl.ds(0, ch)])
        pltpu.sync_copy(dinv_hbm.at[pl.ds(g, ch)], dbuf.at[pl.ds(0, ch)])

        def rb(r, _):
            dv = dbuf[r]
            a0 = abuf[r, pl.ds(0, 16)] * dv
            a1 = abuf[r, pl.ds(16, 16)] * dv
            xbuf[r, pl.ds(0, 16)] = a0
            xbuf[r, pl.ds(16, 16)] = a1
            zbuf[r, pl.ds(0, 16)] = a0 * dv
            zbuf[r, pl.ds(16, 16)] = a1 * dv
            return 0

        lax.fori_loop(0, ch, rb, 0)
        pltpu.sync_copy(xbuf.at[pl.ds(0, ch)], xout_hbm.at[pl.ds(g, ch)])
        pltpu.sync_copy(zbuf.at[pl.ds(0, ch)], zout_hbm.at[pl.ds(g, ch)])

    @pl.when(t < 15)
    def _():
        for k in range(4):
            chunk(t * SLICE + k * 800, 800)

    @pl.when(t == 15)
    def _():
        for k in range(2):
            chunk(t * SLICE + k * 800, 800)
        chunk(t * SLICE + 1600, 400)


def _dinv_body(deg_ref, x0_ref, dinv_ref, z0_ref):
    d = deg_ref[...]
    dv = jnp.where(d > 0, lax.rsqrt(jnp.maximum(d, 1.0)), 0.0)
    dinv_ref[...] = dv
    z0_ref[...] = dv[:, None] * x0_ref[...]


_dinv_call = pl.pallas_call(
    _dinv_body,
    grid=(100,),
    in_specs=[
        pl.BlockSpec((1000,), lambda i: (i,)),
        pl.BlockSpec((1000, EMB), lambda i: (i, 0)),
    ],
    out_specs=[
        pl.BlockSpec((1000,), lambda i: (i,)),
        pl.BlockSpec((1000, EMB), lambda i: (i, 0)),
    ],
    out_shape=[
        jax.ShapeDtypeStruct((NN,), jnp.float32),
        jax.ShapeDtypeStruct((NN, EMB), jnp.float32),
    ],
)


def _mean_body(a_ref, b_ref, c_ref, d_ref, o_ref):
    o_ref[...] = (a_ref[...] + b_ref[...] + c_ref[...] + d_ref[...]) * 0.25


_mean_call = pl.pallas_call(
    _mean_body,
    grid=(50,),
    in_specs=[pl.BlockSpec((2000, EMB), lambda i: (i, 0))] * 4,
    out_specs=pl.BlockSpec((2000, EMB), lambda i: (i, 0)),
    out_shape=jax.ShapeDtypeStruct((NN, EMB), jnp.float32),
)


def kernel(edge_index, user_w, item_w):
    ei = edge_index.astype(jnp.int32)
    npad = EPAD - NEDGE
    src2 = jnp.concatenate(
        [ei[0], jnp.zeros((npad,), jnp.int32)]).reshape(ROWS2D, 128)
    dst2 = jnp.concatenate(
        [ei[1], jnp.full((npad,), NN, jnp.int32)]).reshape(ROWS2D, 128)
    x0 = jnp.concatenate([user_w, item_w], axis=0)

    deg = _deg_kernel(dst2)
    dinv, z = _dinv_call(deg, x0)
    xs = [x0]
    for _ in range(N_LAYERS):
        xnext, z = _prop_kernel(src2, dst2, z, dinv)
        xs.append(xnext)
    y = _mean_call(*xs)
    return (y[:N_USERS], y[N_USERS:])


# speedup vs baseline: 12.0107x; 12.0107x over previous
import functools

import jax
import jax.numpy as jnp
from jax import lax
from jax.experimental import pallas as pl
from jax.experimental.pallas import tpu as pltpu
from jax.experimental.pallas import tpu_sc as plsc

N_USERS = 30000
N_ITEMS = 70000
NN = N_USERS + N_ITEMS
HALF = NN // 2
EMB = 32
NEDGE = 1600000
EPAD = 1638400
ROWS2D = EPAD // 128
TROWS = ROWS2D // 16
WINDOWS = 100
WINDOWS_P = 200
ACC_ROWS = 51200
SLICE = ACC_ROWS // 16
N_LAYERS = 3

_mesh = plsc.VectorSubcoreMesh(core_axis_name="c", subcore_axis_name="s")


@functools.partial(
    pl.kernel,
    out_type=jax.ShapeDtypeStruct((NN,), jnp.float32),
    mesh=_mesh,
    scratch_types=[
        pltpu.VMEM_SHARED((ACC_ROWS,), jnp.float32),
        pltpu.VMEM((8, 128), jnp.int32),
        pltpu.VMEM((8, 128), jnp.int32),
        pltpu.VMEM((8, 128), jnp.float32),
        pltpu.VMEM((800,), jnp.float32),
    ],
    compiler_params=pltpu.CompilerParams(use_tc_tiling_on_sc=False),
)
def _deg_kernel(dst_hbm, deg_hbm, acc, dstb, ldst, ones, zbuf):
    c = lax.axis_index("c")
    t = lax.axis_index("s")
    base = c * HALF
    z16 = jnp.zeros((16,), jnp.float32)
    o16 = jnp.ones((16,), jnp.float32)

    def zfill(i, _):
        zbuf[pl.ds(i * 16, 16)] = z16
        return 0

    lax.fori_loop(0, 50, zfill, 0)
    for r in range(8):
        for q in range(8):
            ones[r, pl.ds(q * 16, 16)] = o16
    for k in range(4):
        pltpu.sync_copy(zbuf, acc.at[pl.ds(t * SLICE + k * 800, 800)])
    plsc.subcore_barrier()

    def body(w, _):
        row = t * TROWS + w * 8
        pltpu.sync_copy(dst_hbm.at[pl.ds(row, 8)], dstb)
        for r in range(8):
            for q in range(8):
                v = dstb[r, pl.ds(q * 16, 16)]
                inr = (v >= base) & (v < base + HALF)
                dump = (HALF + (r * 8 + q) * 16) + lax.iota(jnp.int32, 16)
                ldst[r, pl.ds(q * 16, 16)] = jnp.where(inr, v - base, dump)
        for r in range(8):
            pltpu.sync_copy(ones.at[r], acc.at[ldst.at[r]], add=True)
        return 0

    lax.fori_loop(0, WINDOWS, body, 0)
    plsc.subcore_barrier()

    def wchunk(lo, ch):
        pltpu.sync_copy(acc.at[pl.ds(lo, ch)], zbuf.at[pl.ds(0, ch)])
        pltpu.sync_copy(zbuf.at[pl.ds(0, ch)],
                        deg_hbm.at[pl.ds(base + lo, ch)])

    @pl.when(t < 15)
    def _():
        for k in range(4):
            wchunk(t * SLICE + k * 800, 800)

    @pl.when(t == 15)
    def _():
        for k in range(2):
            wchunk(t * SLICE + k * 800, 800)
        wchunk(t * SLICE + 1600, 400)


@functools.partial(
    pl.kernel,
    out_type=(jax.ShapeDtypeStruct((NN, EMB), jnp.float32),
              jax.ShapeDtypeStruct((NN, EMB), jnp.float32)),
    mesh=_mesh,
    scratch_types=[
        pltpu.VMEM_SHARED((ACC_ROWS, EMB), jnp.float32),
        pltpu.VMEM((4, 128), jnp.int32),
        pltpu.VMEM((4, 128), jnp.int32),
        pltpu.VMEM((4, 128), jnp.int32),
        pltpu.VMEM((512, EMB), jnp.float32),
        pltpu.VMEM((128,), jnp.float32),
        pltpu.SemaphoreType.DMA,
        pltpu.SemaphoreType.DMA,
    ],
    compiler_params=pltpu.CompilerParams(use_tc_tiling_on_sc=False),
)
def _prop_kernel(src_hbm, dst_hbm, zin_hbm, dinv_hbm, xout_hbm, zout_hbm,
                 acc, srcb, dstb, ldst, buf, dbuf, sem, sem2):
    c = lax.axis_index("c")
    t = lax.axis_index("s")
    base = c * HALF
    z16 = jnp.zeros((16,), jnp.float32)

    def zfill(i, _):
        buf[i, pl.ds(0, 16)] = z16
        buf[i, pl.ds(16, 16)] = z16
        return 0

    lax.fori_loop(0, 128, zfill, 0)
    for k in range(25):
        pltpu.sync_copy(buf.at[pl.ds(0, 128)],
                        acc.at[pl.ds(t * SLICE + k * 128, 128)])
    plsc.subcore_barrier()

    def body(w, _):
        row = t * TROWS + w * 4
        pltpu.sync_copy(src_hbm.at[pl.ds(row, 4)], srcb)
        pltpu.sync_copy(dst_hbm.at[pl.ds(row, 4)], dstb)
        handles = [
            pltpu.async_copy(zin_hbm.at[srcb.at[j]],
                             buf.at[pl.ds(j * 128, 128)], sem)
            for j in range(4)
        ]
        for r in range(4):
            for q in range(8):
                v = dstb[r, pl.ds(q * 16, 16)]
                inr = (v >= base) & (v < base + HALF)
                dump = (HALF + (r * 8 + q) * 16) + lax.iota(jnp.int32, 16)
                ldst[r, pl.ds(q * 16, 16)] = jnp.where(inr, v - base, dump)
        for h in handles:
            h.wait()
        shandles = [
            pltpu.async_copy(buf.at[pl.ds(j * 128, 128)],
                             acc.at[ldst.at[j]], sem2, add=True)
            for j in range(4)
        ]
        for h in shandles:
            h.wait()
        return 0

    lax.fori_loop(0, WINDOWS_P, body, 0)
    plsc.subcore_barrier()

    def chunk(lo, ch):
        g = base + lo
        pltpu.sync_copy(acc.at[pl.ds(lo, ch)], buf.at[pl.ds(0, ch)])
        pltpu.sync_copy(dinv_hbm.at[pl.ds(g, ch)], dbuf.at[pl.ds(0, ch)])

        def rb(i, _):
            r0 = i * 16
            dvec = dbuf[pl.ds(r0, 16)]
            for k in range(16):
                dv = dvec[k]
                a0 = buf[r0 + k, pl.ds(0, 16)] * dv
                a1 = buf[r0 + k, pl.ds(16, 16)] * dv
                buf[128 + r0 + k, pl.ds(0, 16)] = a0
                buf[128 + r0 + k, pl.ds(16, 16)] = a1
                buf[256 + r0 + k, pl.ds(0, 16)] = a0 * dv
                buf[256 + r0 + k, pl.ds(16, 16)] = a1 * dv
            return 0

        lax.fori_loop(0, ch // 16, rb, 0)
        pltpu.sync_copy(buf.at[pl.ds(128, ch)], xout_hbm.at[pl.ds(g, ch)])
        pltpu.sync_copy(buf.at[pl.ds(256, ch)], zout_hbm.at[pl.ds(g, ch)])

    @pl.when(t < 15)
    def _():
        for k in range(25):
            chunk(t * SLICE + k * 128, 128)

    @pl.when(t == 15)
    def _():
        for k in range(15):
            chunk(t * SLICE + k * 128, 128)
        chunk(t * SLICE + 15 * 128, 80)


def _dinv_body(deg_ref, x0_ref, dinv_ref, z0_ref):
    d = deg_ref[...]
    dv = jnp.where(d > 0, lax.rsqrt(jnp.maximum(d, 1.0)), 0.0)
    dinv_ref[...] = dv
    z0_ref[...] = dv[:, None] * x0_ref[...]


_dinv_call = pl.pallas_call(
    _dinv_body,
    grid=(98,),
    in_specs=[
        pl.BlockSpec((1024,), lambda i: (i,)),
        pl.BlockSpec((1024, EMB), lambda i: (i, 0)),
    ],
    out_specs=[
        pl.BlockSpec((1024,), lambda i: (i,)),
        pl.BlockSpec((1024, EMB), lambda i: (i, 0)),
    ],
    out_shape=[
        jax.ShapeDtypeStruct((NN,), jnp.float32),
        jax.ShapeDtypeStruct((NN, EMB), jnp.float32),
    ],
)


def _mean_body(a_ref, b_ref, c_ref, d_ref, o_ref):
    o_ref[...] = (a_ref[...] + b_ref[...] + c_ref[...] + d_ref[...]) * 0.25


_mean_call = pl.pallas_call(
    _mean_body,
    grid=(50,),
    in_specs=[pl.BlockSpec((2000, EMB), lambda i: (i, 0))] * 4,
    out_specs=pl.BlockSpec((2000, EMB), lambda i: (i, 0)),
    out_shape=jax.ShapeDtypeStruct((NN, EMB), jnp.float32),
)


def kernel(edge_index, user_w, item_w):
    ei = edge_index.astype(jnp.int32)
    npad = EPAD - NEDGE
    src2 = jnp.concatenate(
        [ei[0], jnp.zeros((npad,), jnp.int32)]).reshape(ROWS2D, 128)
    dst2 = jnp.concatenate(
        [ei[1], jnp.full((npad,), NN, jnp.int32)]).reshape(ROWS2D, 128)
    x0 = jnp.concatenate([user_w, item_w], axis=0)

    deg = _deg_kernel(dst2)
    dinv, z = _dinv_call(deg, x0)
    xs = [x0]
    for _ in range(N_LAYERS):
        xnext, z = _prop_kernel(src2, dst2, z, dinv)
        xs.append(xnext)
    y = _mean_call(*xs)
    return (y[:N_USERS], y[N_USERS:])


# speedup vs baseline: 13.3765x; 1.1137x over previous
import functools

import jax
import jax.numpy as jnp
from jax import lax
from jax.experimental import pallas as pl
from jax.experimental.pallas import tpu as pltpu
from jax.experimental.pallas import tpu_sc as plsc

N_USERS = 30000
N_ITEMS = 70000
NN = N_USERS + N_ITEMS
HALF = NN // 2
EMB = 32
NEDGE = 1600000
EPAD = 1638400
ROWS2D = EPAD // 128
TROWS = ROWS2D // 16
WINDOWS = 100
KPAIR = 200
ACC_ROWS = 51200
SLICE = ACC_ROWS // 16
N_LAYERS = 3

_mesh = plsc.VectorSubcoreMesh(core_axis_name="c", subcore_axis_name="s")


@functools.partial(
    pl.kernel,
    out_type=jax.ShapeDtypeStruct((NN,), jnp.float32),
    mesh=_mesh,
    scratch_types=[
        pltpu.VMEM_SHARED((ACC_ROWS,), jnp.float32),
        pltpu.VMEM((8, 128), jnp.int32),
        pltpu.VMEM((8, 128), jnp.int32),
        pltpu.VMEM((8, 128), jnp.float32),
        pltpu.VMEM((800,), jnp.float32),
    ],
    compiler_params=pltpu.CompilerParams(use_tc_tiling_on_sc=False),
)
def _deg_kernel(dst_hbm, deg_hbm, acc, dstb, ldst, ones, zbuf):
    c = lax.axis_index("c")
    t = lax.axis_index("s")
    base = c * HALF
    z16 = jnp.zeros((16,), jnp.float32)
    o16 = jnp.ones((16,), jnp.float32)

    def zfill(i, _):
        zbuf[pl.ds(i * 16, 16)] = z16
        return 0

    lax.fori_loop(0, 50, zfill, 0)
    for r in range(8):
        for q in range(8):
            ones[r, pl.ds(q * 16, 16)] = o16
    for k in range(4):
        pltpu.sync_copy(zbuf, acc.at[pl.ds(t * SLICE + k * 800, 800)])
    plsc.subcore_barrier()

    def body(w, _):
        row = t * TROWS + w * 8
        pltpu.sync_copy(dst_hbm.at[pl.ds(row, 8)], dstb)
        for r in range(8):
            for q in range(8):
                v = dstb[r, pl.ds(q * 16, 16)]
                inr = (v >= base) & (v < base + HALF)
                dump = (HALF + (r * 8 + q) * 16) + lax.iota(jnp.int32, 16)
                ldst[r, pl.ds(q * 16, 16)] = jnp.where(inr, v - base, dump)
        for r in range(8):
            pltpu.sync_copy(ones.at[r], acc.at[ldst.at[r]], add=True)
        return 0

    lax.fori_loop(0, WINDOWS, body, 0)
    plsc.subcore_barrier()

    def wchunk(lo, ch):
        pltpu.sync_copy(acc.at[pl.ds(lo, ch)], zbuf.at[pl.ds(0, ch)])
        pltpu.sync_copy(zbuf.at[pl.ds(0, ch)],
                        deg_hbm.at[pl.ds(base + lo, ch)])

    @pl.when(t < 15)
    def _():
        for k in range(4):
            wchunk(t * SLICE + k * 800, 800)

    @pl.when(t == 15)
    def _():
        for k in range(2):
            wchunk(t * SLICE + k * 800, 800)
        wchunk(t * SLICE + 1600, 400)


@functools.partial(
    pl.kernel,
    out_type=(jax.ShapeDtypeStruct((NN, EMB), jnp.float32),
              jax.ShapeDtypeStruct((NN, EMB), jnp.float32)),
    mesh=_mesh,
    scratch_types=[
        pltpu.VMEM_SHARED((ACC_ROWS, EMB), jnp.float32),
        pltpu.VMEM((4, 128), jnp.int32),
        pltpu.VMEM((4, 128), jnp.int32),
        pltpu.VMEM((2, 128), jnp.int32),
        pltpu.VMEM((2, 128), jnp.int32),
        pltpu.VMEM((256, EMB), jnp.float32),
        pltpu.VMEM((256, EMB), jnp.float32),
        pltpu.VMEM((128,), jnp.float32),
        pltpu.SemaphoreType.DMA,
        pltpu.SemaphoreType.DMA,
        pltpu.SemaphoreType.DMA,
        pltpu.SemaphoreType.DMA,
    ],
    compiler_params=pltpu.CompilerParams(use_tc_tiling_on_sc=False),
)
def _prop_kernel(il_hbm, zin_hbm, dinv_hbm, xout_hbm, zout_hbm,
                 acc, idxA, idxB, ldA, ldB, rowsA, rowsB, dbuf,
                 semI, semGA, semGB, semS):
    c = lax.axis_index("c")
    t = lax.axis_index("s")
    base = c * HALF
    z16 = jnp.zeros((16,), jnp.float32)

    def zfill(i, _):
        rowsA[i, pl.ds(0, 16)] = z16
        rowsA[i, pl.ds(16, 16)] = z16
        return 0

    lax.fori_loop(0, 128, zfill, 0)
    for k in range(25):
        pltpu.sync_copy(rowsA.at[pl.ds(0, 128)],
                        acc.at[pl.ds(t * SLICE + k * 128, 128)])
    plsc.subcore_barrier()

    iot = lax.iota(jnp.int32, 16)

    def ldcompute(idx, ld):
        for r in range(2):
            for q in range(8):
                v = idx[2 + r, pl.ds(q * 16, 16)]
                inr = (v >= base) & (v < base + HALF)
                dslot = (t * 16 + r * 8 + q) & 63
                dump = (HALF + dslot * 16) + iot
                ld[r, pl.ds(q * 16, 16)] = jnp.where(inr, v - base, dump)

    def gathers(idx, rows, sem):
        return [
            pltpu.async_copy(zin_hbm.at[idx.at[j]],
                             rows.at[pl.ds(j * 128, 128)], sem)
            for j in range(2)
        ]

    def gdrain(idx, rows, sem):
        for j in range(2):
            pltpu.make_async_copy(zin_hbm.at[idx.at[j]],
                                  rows.at[pl.ds(j * 128, 128)], sem).wait()

    def scatters(rows, ld):
        return [
            pltpu.async_copy(rows.at[pl.ds(j * 128, 128)],
                             acc.at[ld.at[j]], semS, add=True)
            for j in range(2)
        ]

    pltpu.sync_copy(il_hbm.at[pl.ds(t * 1600, 4)], idxA)
    gathers(idxA, rowsA, semGA)

    def body(k, _):
        r0 = t * 1600 + k * 8
        hb = pltpu.async_copy(il_hbm.at[pl.ds(r0 + 4, 4)], idxB, semI)
        ldcompute(idxA, ldA)
        hb.wait()
        gb = gathers(idxB, rowsB, semGB)
        gdrain(idxA, rowsA, semGA)
        sa = scatters(rowsA, ldA)

        @pl.when(k < KPAIR - 1)
        def _():
            pltpu.async_copy(il_hbm.at[pl.ds(r0 + 8, 4)], idxA, semI)

        ldcompute(idxB, ldB)
        for h in sa:
            h.wait()

        @pl.when(k < KPAIR - 1)
        def _():
            pltpu.make_async_copy(il_hbm.at[pl.ds(r0 + 8, 4)],
                                  idxA, semI).wait()
            gathers(idxA, rowsA, semGA)

        for h in gb:
            h.wait()
        sb = scatters(rowsB, ldB)
        for h in sb:
            h.wait()
        return 0

    lax.fori_loop(0, KPAIR, body, 0)
    plsc.subcore_barrier()

    def chunk(lo, ch):
        g = base + lo
        pltpu.sync_copy(acc.at[pl.ds(lo, ch)], rowsA.at[pl.ds(0, ch)])
        pltpu.sync_copy(dinv_hbm.at[pl.ds(g, ch)], dbuf.at[pl.ds(0, ch)])

        def rb(i, _):
            r0 = i * 16
            dvec = dbuf[pl.ds(r0, 16)]
            for k in range(16):
                dv = dvec[k]
                a0 = rowsA[r0 + k, pl.ds(0, 16)] * dv
                a1 = rowsA[r0 + k, pl.ds(16, 16)] * dv
                rowsA[128 + r0 + k, pl.ds(0, 16)] = a0
                rowsA[128 + r0 + k, pl.ds(16, 16)] = a1
                rowsB[r0 + k, pl.ds(0, 16)] = a0 * dv
                rowsB[r0 + k, pl.ds(16, 16)] = a1 * dv
            return 0

        lax.fori_loop(0, ch // 16, rb, 0)
        pltpu.sync_copy(rowsA.at[pl.ds(128, ch)], xout_hbm.at[pl.ds(g, ch)])
        pltpu.sync_copy(rowsB.at[pl.ds(0, ch)], zout_hbm.at[pl.ds(g, ch)])

    @pl.when(t < 15)
    def _():
        for k in range(25):
            chunk(t * SLICE + k * 128, 128)

    @pl.when(t == 15)
    def _():
        for k in range(15):
            chunk(t * SLICE + k * 128, 128)
        chunk(t * SLICE + 15 * 128, 80)


def _dinv_body(deg_ref, x0_ref, dinv_ref, z0_ref):
    d = deg_ref[...]
    dv = jnp.where(d > 0, lax.rsqrt(jnp.maximum(d, 1.0)), 0.0)
    dinv_ref[...] = dv
    z0_ref[...] = dv[:, None] * x0_ref[...]


_dinv_call = pl.pallas_call(
    _dinv_body,
    grid=(98,),
    in_specs=[
        pl.BlockSpec((1024,), lambda i: (i,)),
        pl.BlockSpec((1024, EMB), lambda i: (i, 0)),
    ],
    out_specs=[
        pl.BlockSpec((1024,), lambda i: (i,)),
        pl.BlockSpec((1024, EMB), lambda i: (i, 0)),
    ],
    out_shape=[
        jax.ShapeDtypeStruct((NN,), jnp.float32),
        jax.ShapeDtypeStruct((NN, EMB), jnp.float32),
    ],
)


def _mean_body(a_ref, b_ref, c_ref, d_ref, o_ref):
    o_ref[...] = (a_ref[...] + b_ref[...] + c_ref[...] + d_ref[...]) * 0.25


_mean_call = pl.pallas_call(
    _mean_body,
    grid=(50,),
    in_specs=[pl.BlockSpec((2000, EMB), lambda i: (i, 0))] * 4,
    out_specs=pl.BlockSpec((2000, EMB), lambda i: (i, 0)),
    out_shape=jax.ShapeDtypeStruct((NN, EMB), jnp.float32),
)


def kernel(edge_index, user_w, item_w):
    ei = edge_index.astype(jnp.int32)
    npad = EPAD - NEDGE
    src2 = jnp.concatenate(
        [ei[0], jnp.zeros((npad,), jnp.int32)]).reshape(ROWS2D, 128)
    dst2 = jnp.concatenate(
        [ei[1], jnp.full((npad,), NN, jnp.int32)]).reshape(ROWS2D, 128)
    x0 = jnp.concatenate([user_w, item_w], axis=0)
    il = jnp.stack([src2.reshape(6400, 2, 128), dst2.reshape(6400, 2, 128)],
                   axis=1).reshape(25600, 128)

    deg = _deg_kernel(dst2)
    dinv, z = _dinv_call(deg, x0)
    xs = [x0]
    for _ in range(N_LAYERS):
        xnext, z = _prop_kernel(il, z, dinv)
        xs.append(xnext)
    y = _mean_call(*xs)
    return (y[:N_USERS], y[N_USERS:])


# speedup vs baseline: 13.6862x; 1.0232x over previous
import functools

import jax
import jax.numpy as jnp
from jax import lax
from jax.experimental import pallas as pl
from jax.experimental.pallas import tpu as pltpu
from jax.experimental.pallas import tpu_sc as plsc

N_USERS = 30000
N_ITEMS = 70000
NN = N_USERS + N_ITEMS
HALF = NN // 2
EMB = 32
NEDGE = 1600000
EPAD = 1638400
ROWS2D = EPAD // 128
TROWS = ROWS2D // 16
WINDOWS = 100
KPAIR = 200
ACC_ROWS = 51200
SLICE = ACC_ROWS // 16
N_LAYERS = 3

_mesh = plsc.VectorSubcoreMesh(core_axis_name="c", subcore_axis_name="s")


@functools.partial(
    pl.kernel,
    out_type=jax.ShapeDtypeStruct((NN,), jnp.float32),
    mesh=_mesh,
    scratch_types=[
        pltpu.VMEM_SHARED((ACC_ROWS,), jnp.float32),
        pltpu.VMEM((8, 128), jnp.int32),
        pltpu.VMEM((8, 128), jnp.int32),
        pltpu.VMEM((8, 128), jnp.float32),
        pltpu.VMEM((800,), jnp.float32),
    ],
    compiler_params=pltpu.CompilerParams(use_tc_tiling_on_sc=False),
)
def _deg_kernel(dst_hbm, deg_hbm, acc, dstb, ldst, ones, zbuf):
    c = lax.axis_index("c")
    t = lax.axis_index("s")
    base = c * HALF
    z16 = jnp.zeros((16,), jnp.float32)
    o16 = jnp.ones((16,), jnp.float32)

    def zfill(i, _):
        zbuf[pl.ds(i * 16, 16)] = z16
        return 0

    lax.fori_loop(0, 50, zfill, 0)
    for r in range(8):
        for q in range(8):
            ones[r, pl.ds(q * 16, 16)] = o16
    for k in range(4):
        pltpu.sync_copy(zbuf, acc.at[pl.ds(t * SLICE + k * 800, 800)])
    plsc.subcore_barrier()

    def body(w, _):
        row = t * TROWS + w * 8
        pltpu.sync_copy(dst_hbm.at[pl.ds(row, 8)], dstb)
        for r in range(8):
            for q in range(8):
                v = dstb[r, pl.ds(q * 16, 16)]
                inr = (v >= base) & (v < base + HALF)
                dump = (HALF + (r * 8 + q) * 16) + lax.iota(jnp.int32, 16)
                ldst[r, pl.ds(q * 16, 16)] = jnp.where(inr, v - base, dump)
        for r in range(8):
            pltpu.sync_copy(ones.at[r], acc.at[ldst.at[r]], add=True)
        return 0

    lax.fori_loop(0, WINDOWS, body, 0)
    plsc.subcore_barrier()

    def wchunk(lo, ch):
        pltpu.sync_copy(acc.at[pl.ds(lo, ch)], zbuf.at[pl.ds(0, ch)])
        pltpu.sync_copy(zbuf.at[pl.ds(0, ch)],
                        deg_hbm.at[pl.ds(base + lo, ch)])

    @pl.when(t < 15)
    def _():
        for k in range(4):
            wchunk(t * SLICE + k * 800, 800)

    @pl.when(t == 15)
    def _():
        for k in range(2):
            wchunk(t * SLICE + k * 800, 800)
        wchunk(t * SLICE + 1600, 400)


@functools.partial(
    pl.kernel,
    out_type=(jax.ShapeDtypeStruct((NN, EMB), jnp.float32),
              jax.ShapeDtypeStruct((NN, 16), jnp.int32)),
    mesh=_mesh,
    scratch_types=[
        pltpu.VMEM_SHARED((ACC_ROWS, EMB), jnp.float32),
        pltpu.VMEM((4, 128), jnp.int32),
        pltpu.VMEM((4, 128), jnp.int32),
        pltpu.VMEM((2, 128), jnp.int32),
        pltpu.VMEM((2, 128), jnp.int32),
        pltpu.VMEM((256, 16), jnp.int32),
        pltpu.VMEM((256, 16), jnp.int32),
        pltpu.VMEM((256, EMB), jnp.float32),
        pltpu.VMEM((256, EMB), jnp.float32),
        pltpu.VMEM((128,), jnp.float32),
        pltpu.SemaphoreType.DMA,
        pltpu.SemaphoreType.DMA,
        pltpu.SemaphoreType.DMA,
        pltpu.SemaphoreType.DMA,
    ],
    compiler_params=pltpu.CompilerParams(use_tc_tiling_on_sc=False,
                                         needs_layout_passes=False),
)
def _prop_kernel(il_hbm, zin_hbm, dinv_hbm, xout_hbm, zout_hbm,
                 acc, idxA, idxB, ldA, ldB, prowsA, prowsB, rowsA, rowsB,
                 dbuf, semI, semGA, semGB, semS):
    c = lax.axis_index("c")
    t = lax.axis_index("s")
    base = c * HALF
    z16 = jnp.zeros((16,), jnp.float32)

    def zfill(i, _):
        rowsA[i, pl.ds(0, 16)] = z16
        rowsA[i, pl.ds(16, 16)] = z16
        return 0

    lax.fori_loop(0, 128, zfill, 0)

    def zcp(k, _):
        pltpu.sync_copy(rowsA.at[pl.ds(0, 128)],
                        acc.at[pl.ds(t * SLICE + k * 128, 128)])
        return 0

    lax.fori_loop(0, 25, zcp, 0)
    plsc.subcore_barrier()

    iot = lax.iota(jnp.int32, 16)
    himask = jnp.full((16,), -65536, jnp.int32)

    def ldcompute(idx, ld):
        for r in range(2):
            for q in range(8):
                v = idx[2 + r, pl.ds(q * 16, 16)]
                inr = (v >= base) & (v < base + HALF)
                dslot = (t * 16 + r * 8 + q) & 63
                dump = (HALF + dslot * 16) + iot
                ld[r, pl.ds(q * 16, 16)] = jnp.where(inr, v - base, dump)

    def unpack_rows(prows, rows):
        def ub(r, _):
            w = prows[r, pl.ds(0, 16)]
            rows[r, pl.ds(0, 16)] = plsc.bitcast(w << 16, jnp.float32)
            rows[r, pl.ds(16, 16)] = plsc.bitcast(w & himask, jnp.float32)
            return 0

        lax.fori_loop(0, 256, ub, 0)

    def gathers(idx, prows, sem):
        return [
            pltpu.async_copy(zin_hbm.at[idx.at[j]],
                             prows.at[pl.ds(j * 128, 128)], sem)
            for j in range(2)
        ]

    def gdrain(idx, prows, sem):
        for j in range(2):
            pltpu.make_async_copy(zin_hbm.at[idx.at[j]],
                                  prows.at[pl.ds(j * 128, 128)], sem).wait()

    def scatters(rows, ld):
        return [
            pltpu.async_copy(rows.at[pl.ds(j * 128, 128)],
                             acc.at[ld.at[j]], semS, add=True)
            for j in range(2)
        ]

    pltpu.sync_copy(il_hbm.at[pl.ds(t * 1600, 4)], idxA)
    gathers(idxA, prowsA, semGA)

    def body(k, _):
        r0 = t * 1600 + k * 8
        hb = pltpu.async_copy(il_hbm.at[pl.ds(r0 + 4, 4)], idxB, semI)
        ldcompute(idxA, ldA)
        hb.wait()
        gb = gathers(idxB, prowsB, semGB)
        gdrain(idxA, prowsA, semGA)
        unpack_rows(prowsA, rowsA)
        sa = scatters(rowsA, ldA)

        @pl.when(k < KPAIR - 1)
        def _():
            pltpu.async_copy(il_hbm.at[pl.ds(r0 + 8, 4)], idxA, semI)

        ldcompute(idxB, ldB)
        for h in sa:
            h.wait()

        @pl.when(k < KPAIR - 1)
        def _():
            pltpu.make_async_copy(il_hbm.at[pl.ds(r0 + 8, 4)],
                                  idxA, semI).wait()
            gathers(idxA, prowsA, semGA)

        for h in gb:
            h.wait()
        unpack_rows(prowsB, rowsB)
        sb = scatters(rowsB, ldB)
        for h in sb:
            h.wait()
        return 0

    lax.fori_loop(0, KPAIR, body, 0)
    plsc.subcore_barrier()

    rnd = jnp.full((16,), 32768, jnp.int32)

    def chunk(lo, ch):
        g = base + lo
        pltpu.sync_copy(acc.at[pl.ds(lo, ch)], rowsA.at[pl.ds(0, ch)])
        pltpu.sync_copy(dinv_hbm.at[pl.ds(g, ch)], dbuf.at[pl.ds(0, ch)])

        def rb(i, _):
            r0 = i * 16
            dvec = dbuf[pl.ds(r0, 16)]
            for k in range(16):
                dv = dvec[k]
                a0 = rowsA[r0 + k, pl.ds(0, 16)] * dv
                a1 = rowsA[r0 + k, pl.ds(16, 16)] * dv
                rowsA[128 + r0 + k, pl.ds(0, 16)] = a0
                rowsA[128 + r0 + k, pl.ds(16, 16)] = a1
                z0 = a0 * dv
                z1 = a1 * dv
                w = (((plsc.bitcast(z0, jnp.int32) + rnd) >> 16) & 65535) | \
                    ((plsc.bitcast(z1, jnp.int32) + rnd) & himask)
                prowsB[r0 + k, pl.ds(0, 16)] = w
            return 0

        lax.fori_loop(0, ch // 16, rb, 0)
        pltpu.sync_copy(rowsA.at[pl.ds(128, ch)], xout_hbm.at[pl.ds(g, ch)])
        pltpu.sync_copy(prowsB.at[pl.ds(0, ch)], zout_hbm.at[pl.ds(g, ch)])

    @pl.when(t < 15)
    def _():
        def pc(k, _):
            chunk(t * SLICE + k * 128, 128)
            return 0

        lax.fori_loop(0, 25, pc, 0)

    @pl.when(t == 15)
    def _():
        def pc(k, _):
            chunk(t * SLICE + k * 128, 128)
            return 0

        lax.fori_loop(0, 15, pc, 0)
        chunk(t * SLICE + 15 * 128, 80)


def _dinv_body(deg_ref, x0_ref, dinv_ref, z0_ref):
    d = deg_ref[...]
    dv = jnp.where(d > 0, lax.rsqrt(jnp.maximum(d, 1.0)), 0.0)
    dinv_ref[...] = dv
    z0_ref[...] = dv[:, None] * x0_ref[...]


_dinv_call = pl.pallas_call(
    _dinv_body,
    grid=(98,),
    in_specs=[
        pl.BlockSpec((1024,), lambda i: (i,)),
        pl.BlockSpec((1024, EMB), lambda i: (i, 0)),
    ],
    out_specs=[
        pl.BlockSpec((1024,), lambda i: (i,)),
        pl.BlockSpec((1024, EMB), lambda i: (i, 0)),
    ],
    out_shape=[
        jax.ShapeDtypeStruct((NN,), jnp.float32),
        jax.ShapeDtypeStruct((NN, EMB), jnp.float32),
    ],
)


def _mean_body(a_ref, b_ref, c_ref, d_ref, o_ref):
    o_ref[...] = (a_ref[...] + b_ref[...] + c_ref[...] + d_ref[...]) * 0.25


_mean_call = pl.pallas_call(
    _mean_body,
    grid=(50,),
    in_specs=[pl.BlockSpec((2000, EMB), lambda i: (i, 0))] * 4,
    out_specs=pl.BlockSpec((2000, EMB), lambda i: (i, 0)),
    out_shape=jax.ShapeDtypeStruct((NN, EMB), jnp.float32),
)


def kernel(edge_index, user_w, item_w):
    ei = edge_index.astype(jnp.int32)
    npad = EPAD - NEDGE
    src2 = jnp.concatenate(
        [ei[0], jnp.zeros((npad,), jnp.int32)]).reshape(ROWS2D, 128)
    dst2 = jnp.concatenate(
        [ei[1], jnp.full((npad,), NN, jnp.int32)]).reshape(ROWS2D, 128)
    x0 = jnp.concatenate([user_w, item_w], axis=0)
    il = jnp.stack([src2.reshape(6400, 2, 128), dst2.reshape(6400, 2, 128)],
                   axis=1).reshape(25600, 128)

    deg = _deg_kernel(dst2)
    dinv, z0 = _dinv_call(deg, x0)
    z = lax.bitcast_convert_type(
        z0.astype(jnp.bfloat16).reshape(NN, 16, 2), jnp.int32)
    x0p = jnp.concatenate([x0[:, 0::2], x0[:, 1::2]], axis=1)
    xs = [x0p]
    for _ in range(N_LAYERS):
        xnext, z = _prop_kernel(il, z, dinv)
        xs.append(xnext)
    y = _mean_call(*xs)
    yt = jnp.stack([y[:, :16], y[:, 16:]], axis=2).reshape(NN, EMB)
    return (yt[:N_USERS], yt[N_USERS:])
